# baseline (speedup 1.0000x reference)
import functools

import jax
import jax.numpy as jnp
from jax import lax
from jax.experimental import pallas as pl
from jax.experimental.pallas import tpu as pltpu
from jax.experimental.pallas import tpu_sc as plsc

EMBED_DIM = 32
BATCH = 16384
VOCAB = 1_000_000

_info = plsc.get_sparse_core_info()
_NC, _NS = _info.num_cores, _info.num_subcores
_NW = _NC * _NS

_NTC_FULL = VOCAB // 128
_TAIL = VOCAB - _NTC_FULL * 128
_ROWS = (_NTC_FULL + 1) * EMBED_DIM
_WLIN_WORDS = _ROWS * 128
_IGNORE = -1


_CH = 6
_NCH = _NTC_FULL // _CH


def _make_reformat(D):
    mesh = plsc.VectorSubcoreMesh(core_axis_name="c", subcore_axis_name="s")
    n_iter = (_NCH + _NW - 1) // _NW

    @functools.partial(
        pl.kernel,
        mesh=mesh,
        out_type=jax.ShapeDtypeStruct((_ROWS, 128), jnp.float32),
        scratch_types=[
            pltpu.VMEM((3, D, _CH * 128), jnp.float32),
            pltpu.VMEM((D, _TAIL), jnp.float32),
            pltpu.SemaphoreType.DMA,
            pltpu.SemaphoreType.DMA,
        ],
    )
    def reformat_kernel(wt_hbm, wlin_hbm, bufs, tail_buf, sem_r, sem_w):
        wid = lax.axis_index("s") * _NC + lax.axis_index("c")

        def ch_of(i):
            return wid + i * _NW

        def read(i, slot):
            ch = ch_of(i)

            @pl.when(ch < _NCH)
            def _():
                pltpu.async_copy(
                    wt_hbm.at[:, pl.ds(ch * _CH * 128, _CH * 128)],
                    bufs.at[slot],
                    sem_r,
                )

        def wait_read(i, slot):
            ch = ch_of(i)

            @pl.when(ch < _NCH)
            def _():
                pltpu.make_async_copy(
                    wt_hbm.at[:, pl.ds(0, _CH * 128)], bufs.at[slot], sem_r
                ).wait()

        def write(i, slot):
            ch = ch_of(i)

            @pl.when(ch < _NCH)
            def _():
                for q in range(_CH):
                    pltpu.async_copy(
                        bufs.at[slot, :, pl.ds(q * 128, 128)],
                        wlin_hbm.at[pl.ds((ch * _CH + q) * D, D), :],
                        sem_w,
                    )

        def wait_write(i, slot):
            ch = ch_of(i)

            @pl.when(ch < _NCH)
            def _():
                for _q in range(_CH):
                    pltpu.make_async_copy(
                        bufs.at[slot, :, pl.ds(0, 128)],
                        wlin_hbm.at[pl.ds(0, D), :],
                        sem_w,
                    ).wait()

        @pl.when(wid == 0)
        def _():
            pltpu.sync_copy(
                wt_hbm.at[:, pl.ds(_NTC_FULL * 128, _TAIL)], tail_buf
            )
            for r in range(D):
                for q in range(_TAIL // 16):
                    bufs[2, r, pl.ds(q * 16, 16)] = tail_buf[r, pl.ds(q * 16, 16)]
            pltpu.sync_copy(
                bufs.at[2, :, pl.ds(0, 128)],
                wlin_hbm.at[pl.ds(_NTC_FULL * D, D), :],
            )

        read(0, 0)
        read(1, 1)

        def body(i, _):
            slot = lax.rem(i, 3)
            wait_read(i, slot)
            write(i, slot)

            @pl.when(i >= 1)
            def _():
                wait_write(i - 1, lax.rem(i - 1, 3))

            @pl.when(i + 2 < n_iter)
            def _():
                read(i + 2, lax.rem(i + 2, 3))

            return 0

        lax.fori_loop(0, n_iter, body, 0, unroll=1)
        wait_write(n_iter - 1, lax.rem(n_iter - 1, 3))

    return reformat_kernel


def _make_gather(D, B):
    b_per_w = B // _NW
    mesh = plsc.VectorSubcoreMesh(core_axis_name="c", subcore_axis_name="s")

    @functools.partial(
        pl.kernel,
        mesh=mesh,
        out_type=jax.ShapeDtypeStruct((D, B), jnp.float32),
        scratch_types=[
            pltpu.VMEM((b_per_w,), jnp.int32),
            pltpu.VMEM((b_per_w,), jnp.int32),
            pltpu.VMEM((D, b_per_w), jnp.float32),
            pltpu.SemaphoreType.DMA,
        ],
        compiler_params=pltpu.CompilerParams(use_tc_tiling_on_sc=False),
    )
    def gather_kernel(idx_hbm, wlin_hbm, out_hbm, idx_v, main_v, stage_v, sem):
        wid = lax.axis_index("s") * _NC + lax.axis_index("c")
        base = wid * b_per_w

        pltpu.sync_copy(idx_hbm.at[pl.ds(base, b_per_w)], idx_v)

        def prep(k, _):
            v = idx_v[pl.ds(k * 16, 16)]
            main_v[pl.ds(k * 16, 16)] = (
                lax.shift_left(lax.shift_right_logical(v, 7), 12)
                + lax.bitwise_and(v, 127)
            )
            return 0

        lax.fori_loop(0, b_per_w // 16, prep, 0, unroll=1)

        for c in range(D):
            pltpu.async_copy(
                wlin_hbm.at[pl.ds(c * 128, _WLIN_WORDS - c * 128)].at[main_v],
                stage_v.at[c],
                sem,
            )
        for c in range(D):
            pltpu.make_async_copy(
                wlin_hbm.at[pl.ds(0, _WLIN_WORDS)].at[main_v],
                stage_v.at[c],
                sem,
            ).wait()

        pltpu.sync_copy(stage_v, out_hbm.at[:, pl.ds(base, b_per_w)])

    return gather_kernel


_reformat = _make_reformat(EMBED_DIM)
_gather = _make_gather(EMBED_DIM, BATCH)


@jax.jit
def kernel(indices, weight):
    wlin = _reformat(weight.T)
    out_t = _gather(indices.astype(jnp.int32), wlin.reshape(-1))
    return out_t.T


# speedup vs baseline: 1.0089x; 1.0089x over previous
import functools

import jax
import jax.numpy as jnp
from jax import lax
from jax.experimental import pallas as pl
from jax.experimental.pallas import tpu as pltpu
from jax.experimental.pallas import tpu_sc as plsc

EMBED_DIM = 32
BATCH = 16384
VOCAB = 1_000_000

_info = plsc.get_sparse_core_info()
_NC, _NS = _info.num_cores, _info.num_subcores
_NW = _NC * _NS

_NTC_FULL = VOCAB // 128
_TAIL = VOCAB - _NTC_FULL * 128
_ROWS = (_NTC_FULL + 1) * EMBED_DIM
_WLIN_WORDS = _ROWS * 128
_IGNORE = -1


_CH = 6
_NCH = _NTC_FULL // _CH


def _make_reformat(D):
    mesh = plsc.VectorSubcoreMesh(core_axis_name="c", subcore_axis_name="s")
    n_iter = (_NCH + _NW - 1) // _NW

    @functools.partial(
        pl.kernel,
        mesh=mesh,
        out_type=jax.ShapeDtypeStruct((_ROWS, 128), jnp.float32),
        scratch_types=[
            pltpu.VMEM((4, D, _CH * 128), jnp.float32),
            pltpu.VMEM((D, _TAIL), jnp.float32),
            pltpu.SemaphoreType.DMA,
            pltpu.SemaphoreType.DMA,
        ],
    )
    def reformat_kernel(wt_hbm, wlin_hbm, bufs, tail_buf, sem_r, sem_w):
        wid = lax.axis_index("s") * _NC + lax.axis_index("c")

        def ch_of(i):
            return wid + i * _NW

        def read(i, slot):
            ch = ch_of(i)

            @pl.when(ch < _NCH)
            def _():
                pltpu.async_copy(
                    wt_hbm.at[:, pl.ds(ch * _CH * 128, _CH * 128)],
                    bufs.at[slot],
                    sem_r,
                )

        def wait_read(i, slot):
            ch = ch_of(i)

            @pl.when(ch < _NCH)
            def _():
                pltpu.make_async_copy(
                    wt_hbm.at[:, pl.ds(0, _CH * 128)], bufs.at[slot], sem_r
                ).wait()

        def write(i, slot):
            ch = ch_of(i)

            @pl.when(ch < _NCH)
            def _():
                for q in range(_CH):
                    pltpu.async_copy(
                        bufs.at[slot, :, pl.ds(q * 128, 128)],
                        wlin_hbm.at[pl.ds((ch * _CH + q) * D, D), :],
                        sem_w,
                    )

        def wait_write(i, slot):
            ch = ch_of(i)

            @pl.when(ch < _NCH)
            def _():
                for _q in range(_CH):
                    pltpu.make_async_copy(
                        bufs.at[slot, :, pl.ds(0, 128)],
                        wlin_hbm.at[pl.ds(0, D), :],
                        sem_w,
                    ).wait()

        @pl.when(wid == 0)
        def _():
            pltpu.sync_copy(
                wt_hbm.at[:, pl.ds(_NTC_FULL * 128, _TAIL)], tail_buf
            )
            for r in range(D):
                for q in range(_TAIL // 16):
                    bufs[3, r, pl.ds(q * 16, 16)] = tail_buf[r, pl.ds(q * 16, 16)]
            pltpu.sync_copy(
                bufs.at[3, :, pl.ds(0, 128)],
                wlin_hbm.at[pl.ds(_NTC_FULL * D, D), :],
            )

        read(0, 0)
        read(1, 1)
        read(2, 2)

        def body(i, _):
            slot = lax.rem(i, 4)
            wait_read(i, slot)
            write(i, slot)

            @pl.when(i >= 1)
            def _():
                wait_write(i - 1, lax.rem(i - 1, 4))

            @pl.when(i + 3 < n_iter)
            def _():
                read(i + 3, lax.rem(i + 3, 4))

            return 0

        lax.fori_loop(0, n_iter, body, 0, unroll=1)
        wait_write(n_iter - 1, lax.rem(n_iter - 1, 4))

    return reformat_kernel


def _make_gather(D, B):
    b_per_w = B // _NW
    mesh = plsc.VectorSubcoreMesh(core_axis_name="c", subcore_axis_name="s")

    @functools.partial(
        pl.kernel,
        mesh=mesh,
        out_type=jax.ShapeDtypeStruct((D, B), jnp.float32),
        scratch_types=[
            pltpu.VMEM((b_per_w,), jnp.int32),
            pltpu.VMEM((b_per_w,), jnp.int32),
            pltpu.VMEM((D, b_per_w), jnp.float32),
            pltpu.SemaphoreType.DMA,
        ],
        compiler_params=pltpu.CompilerParams(use_tc_tiling_on_sc=False),
    )
    def gather_kernel(idx_hbm, wlin_hbm, out_hbm, idx_v, main_v, stage_v, sem):
        wid = lax.axis_index("s") * _NC + lax.axis_index("c")
        base = wid * b_per_w

        pltpu.sync_copy(idx_hbm.at[pl.ds(base, b_per_w)], idx_v)

        def prep(k, _):
            v = idx_v[pl.ds(k * 16, 16)]
            main_v[pl.ds(k * 16, 16)] = (
                lax.shift_left(lax.shift_right_logical(v, 7), 12)
                + lax.bitwise_and(v, 127)
            )
            return 0

        lax.fori_loop(0, b_per_w // 16, prep, 0, unroll=1)

        for c in range(D):
            pltpu.async_copy(
                wlin_hbm.at[pl.ds(c * 128, _WLIN_WORDS - c * 128)].at[main_v],
                stage_v.at[c],
                sem,
            )
        for c in range(D):
            pltpu.make_async_copy(
                wlin_hbm.at[pl.ds(0, _WLIN_WORDS)].at[main_v],
                stage_v.at[c],
                sem,
            ).wait()

        pltpu.sync_copy(stage_v, out_hbm.at[:, pl.ds(base, b_per_w)])

    return gather_kernel


_reformat = _make_reformat(EMBED_DIM)
_gather = _make_gather(EMBED_DIM, BATCH)


@jax.jit
def kernel(indices, weight):
    wlin = _reformat(weight.T)
    out_t = _gather(indices.astype(jnp.int32), wlin.reshape(-1))
    return out_t.T
